# baseline (speedup 1.0000x reference)
import functools

import jax
import jax.numpy as jnp
from jax import lax
from jax.experimental import pallas as pl
from jax.experimental.pallas import tpu as pltpu
from jax.experimental.pallas import tpu_sc as plsc

N_STEPS = 1000
BATCH = 16384

_info = plsc.get_sparse_core_info()
_NC, _NS = _info.num_cores, _info.num_subcores
_NW = _NC * _NS
_B_PER_W = BATCH // _NW

_mesh = plsc.VectorSubcoreMesh(core_axis_name="c", subcore_axis_name="s")


@functools.partial(
    pl.kernel,
    mesh=_mesh,
    out_type=(
        jax.ShapeDtypeStruct((BATCH,), jnp.float32),
        jax.ShapeDtypeStruct((BATCH,), jnp.float32),
    ),
    scratch_types=[
        pltpu.VMEM((_B_PER_W,), jnp.int32),
        pltpu.VMEM((_B_PER_W,), jnp.float32),
        pltpu.VMEM((_B_PER_W,), jnp.float32),
        pltpu.VMEM_SHARED((N_STEPS,), jnp.float32),
        pltpu.VMEM_SHARED((N_STEPS,), jnp.float32),
        pltpu.SemaphoreType.DMA,
        pltpu.SemaphoreType.DMA,
        pltpu.SemaphoreType.DMA,
    ],
)
def _noise_lookup(t_hbm, beta_hbm, alpha_hbm, bt_hbm, at_hbm,
                  idx_v, ob_v, oa_v, beta_s, alpha_s, sem_i, sem_g, sem_o):
    wid = lax.axis_index("s") * _NC + lax.axis_index("c")
    base = wid * _B_PER_W
    idx_c = pltpu.async_copy(t_hbm.at[pl.ds(base, _B_PER_W)], idx_v, sem_i)
    b_c = pltpu.async_copy(beta_hbm, beta_s, sem_i)
    a_c = pltpu.async_copy(alpha_hbm, alpha_s, sem_i)
    idx_c.wait()
    b_c.wait()
    a_c.wait()
    cb = pltpu.async_copy(beta_s.at[idx_v], ob_v, sem_g)
    ca = pltpu.async_copy(alpha_s.at[idx_v], oa_v, sem_g)
    cb.wait()
    ob_c = pltpu.async_copy(ob_v, bt_hbm.at[pl.ds(base, _B_PER_W)], sem_o)
    ca.wait()
    oa_c = pltpu.async_copy(oa_v, at_hbm.at[pl.ds(base, _B_PER_W)], sem_o)
    ob_c.wait()
    oa_c.wait()


def kernel(t, beta, alpha):
    return _noise_lookup(t, beta, alpha)


# speedup vs baseline: 1.0687x; 1.0687x over previous
import functools

import jax
import jax.numpy as jnp
from jax import lax
from jax.experimental import pallas as pl
from jax.experimental.pallas import tpu as pltpu
from jax.experimental.pallas import tpu_sc as plsc

N_STEPS = 1000
BATCH = 16384

_info = plsc.get_sparse_core_info()
_NS = _info.num_subcores
_B_PER_W = BATCH // _NS

_mesh = plsc.VectorSubcoreMesh(core_axis_name="c", subcore_axis_name="s",
                               num_cores=1)


@functools.partial(
    pl.kernel,
    mesh=_mesh,
    out_type=(
        jax.ShapeDtypeStruct((BATCH,), jnp.float32),
        jax.ShapeDtypeStruct((BATCH,), jnp.float32),
    ),
    scratch_types=[
        pltpu.VMEM((_B_PER_W,), jnp.int32),
        pltpu.VMEM((_B_PER_W,), jnp.float32),
        pltpu.VMEM((_B_PER_W,), jnp.float32),
        pltpu.VMEM_SHARED((N_STEPS,), jnp.float32),
        pltpu.VMEM_SHARED((N_STEPS,), jnp.float32),
        pltpu.SemaphoreType.DMA,
        pltpu.SemaphoreType.DMA,
        pltpu.SemaphoreType.DMA,
    ],
)
def _noise_lookup(t_hbm, beta_hbm, alpha_hbm, bt_hbm, at_hbm,
                  idx_v, ob_v, oa_v, beta_s, alpha_s, sem_i, sem_g, sem_o):
    sid = lax.axis_index("s")
    base = sid * _B_PER_W
    idx_c = pltpu.async_copy(t_hbm.at[pl.ds(base, _B_PER_W)], idx_v, sem_i)
    b_c = pltpu.async_copy(beta_hbm, beta_s, sem_i)
    a_c = pltpu.async_copy(alpha_hbm, alpha_s, sem_i)
    idx_c.wait()
    b_c.wait()
    a_c.wait()
    half = _B_PER_W // 2
    gathers = []
    for h in range(2):
        hs = pl.ds(h * half, half)
        gathers.append((pltpu.async_copy(beta_s.at[idx_v.at[hs]], ob_v.at[hs], sem_g),
                        pltpu.async_copy(alpha_s.at[idx_v.at[hs]], oa_v.at[hs], sem_g)))
    outs = []
    for h, (gb, ga) in enumerate(gathers):
        hs = pl.ds(h * half, half)
        gb.wait()
        outs.append(pltpu.async_copy(ob_v.at[hs], bt_hbm.at[pl.ds(base + h * half, half)], sem_o))
        ga.wait()
        outs.append(pltpu.async_copy(oa_v.at[hs], at_hbm.at[pl.ds(base + h * half, half)], sem_o))
    for c in outs:
        c.wait()


def kernel(t, beta, alpha):
    return _noise_lookup(t, beta, alpha)
